# baseline (speedup 1.0000x reference)
import functools

import jax
import jax.numpy as jnp
from jax import lax
from jax.experimental import pallas as pl
from jax.experimental.pallas import tpu as pltpu


def _conv3x3_kernel(x_ref, w_ref, b_ref, o_ref, s0_ref, s1_ref, *,
                    wdim, hw, base, nimg):
    cin = x_ref.shape[1]
    ls = s0_ref.shape[1]
    scratches = (s0_ref, s1_ref)

    col = lax.broadcasted_iota(jnp.int32, (1, hw), 1) % wdim
    m_r = col == wdim - 1
    m_l = col == 0
    zl0, zl1 = base - 2 * wdim, base + 2 * wdim
    zr0, zr1 = base + hw - 2 * wdim, min(base + hw + 2 * wdim, ls)

    for s in scratches:
        s[:, zl0:zl1] = jnp.zeros((3 * cin, zl1 - zl0), jnp.bfloat16)
        s[:, zr0:zr1] = jnp.zeros((3 * cin, zr1 - zr0), jnp.bfloat16)

    def build(b, s_ref):
        xc = x_ref[b].astype(jnp.bfloat16)
        x_mr = jnp.where(m_r, jnp.bfloat16(0), xc)
        x_ml = jnp.where(m_l, jnp.bfloat16(0), xc)
        srcs = (x_mr, xc, x_ml)
        for kw in range(3):
            p = base - (kw - 1)
            s_ref[kw * cin:(kw + 1) * cin, p:p + hw] = srcs[kw]

    build(0, scratches[0])
    for b in range(nimg):
        s_ref = scratches[b % 2]
        if b + 1 < nimg:
            build(b + 1, scratches[(b + 1) % 2])
        acc = b_ref[...].astype(jnp.float32)
        acc = jnp.broadcast_to(acc, (w_ref.shape[1], hw))
        for kh in range(3):
            t = base + (kh - 1) * wdim
            acc = acc + jnp.dot(w_ref[kh], s_ref[:, t:t + hw],
                                preferred_element_type=jnp.float32)
        o_ref[b] = jnp.maximum(acc, 0.0)


def kernel(x, weight, bias):
    n, cin, h, w = x.shape
    cout = weight.shape[0]
    hw = h * w
    base = 128
    ls = (base + w + 1 + hw + 127) // 128 * 128
    nimg = 4

    x3 = x.reshape(n // nimg, nimg, cin, hw)
    wk = jnp.transpose(weight, (2, 0, 3, 1)).reshape(3, cout, 3 * cin)
    wk = wk.astype(jnp.bfloat16)
    b2 = bias.astype(jnp.float32).reshape(cout, 1)

    body = functools.partial(_conv3x3_kernel, wdim=w, hw=hw, base=base,
                             nimg=nimg)
    out = pl.pallas_call(
        body,
        out_shape=jax.ShapeDtypeStruct((n // nimg, nimg, cout, hw),
                                       jnp.float32),
        grid=(n // nimg,),
        in_specs=[
            pl.BlockSpec((None, nimg, cin, hw), lambda i: (i, 0, 0, 0)),
            pl.BlockSpec((3, cout, 3 * cin), lambda i: (0, 0, 0)),
            pl.BlockSpec((cout, 1), lambda i: (0, 0)),
        ],
        out_specs=pl.BlockSpec((None, nimg, cout, hw), lambda i: (i, 0, 0, 0)),
        scratch_shapes=[pltpu.VMEM((3 * cin, ls), jnp.bfloat16),
                        pltpu.VMEM((3 * cin, ls), jnp.bfloat16)],
        compiler_params=pltpu.CompilerParams(
            dimension_semantics=("parallel",),
            vmem_limit_bytes=60 * 1024 * 1024),
    )(x3, wk, b2)
    return out.reshape(n, cout, h, w)


# speedup vs baseline: 1.0001x; 1.0001x over previous
import functools

import jax
import jax.numpy as jnp
from jax import lax
from jax.experimental import pallas as pl
from jax.experimental.pallas import tpu as pltpu


def _conv3x3_kernel(x_ref, w_ref, b_ref, o_ref, s0_ref, s1_ref, s2_ref, *,
                    wdim, hw, base, nimg):
    cin = x_ref.shape[1]
    ls = s0_ref.shape[1]
    scratches = (s0_ref, s1_ref, s2_ref)

    col = lax.broadcasted_iota(jnp.int32, (1, hw), 1) % wdim
    m_r = col == wdim - 1
    m_l = col == 0
    zl0, zl1 = base - 2 * wdim, base + 2 * wdim
    zr0, zr1 = base + hw - 2 * wdim, min(base + hw + 2 * wdim, ls)

    for s in scratches:
        s[:, zl0:zl1] = jnp.zeros((3 * cin, zl1 - zl0), jnp.bfloat16)
        s[:, zr0:zr1] = jnp.zeros((3 * cin, zr1 - zr0), jnp.bfloat16)

    def build(b, s_ref):
        xc = x_ref[b].astype(jnp.bfloat16)
        x_mr = jnp.where(m_r, jnp.bfloat16(0), xc)
        x_ml = jnp.where(m_l, jnp.bfloat16(0), xc)
        srcs = (x_mr, xc, x_ml)
        for kw in range(3):
            p = base - (kw - 1)
            s_ref[kw * cin:(kw + 1) * cin, p:p + hw] = srcs[kw]

    build(0, scratches[0])
    for b in range(nimg):
        s_ref = scratches[b % 3]
        if b + 1 < nimg:
            build(b + 1, scratches[(b + 1) % 3])
        acc = b_ref[...].astype(jnp.float32)
        acc = jnp.broadcast_to(acc, (w_ref.shape[1], hw))
        for kh in range(3):
            t = base + (kh - 1) * wdim
            acc = acc + jnp.dot(w_ref[kh], s_ref[:, t:t + hw],
                                preferred_element_type=jnp.float32)
        o_ref[b] = jnp.maximum(acc, 0.0)


def kernel(x, weight, bias):
    n, cin, h, w = x.shape
    cout = weight.shape[0]
    hw = h * w
    base = 128
    ls = (base + w + 1 + hw + 127) // 128 * 128
    nimg = 8

    x3 = x.reshape(n // nimg, nimg, cin, hw)
    wk = jnp.transpose(weight, (2, 0, 3, 1)).reshape(3, cout, 3 * cin)
    wk = wk.astype(jnp.bfloat16)
    b2 = bias.astype(jnp.float32).reshape(cout, 1)

    body = functools.partial(_conv3x3_kernel, wdim=w, hw=hw, base=base,
                             nimg=nimg)
    out = pl.pallas_call(
        body,
        out_shape=jax.ShapeDtypeStruct((n // nimg, nimg, cout, hw),
                                       jnp.float32),
        grid=(n // nimg,),
        in_specs=[
            pl.BlockSpec((None, nimg, cin, hw), lambda i: (i, 0, 0, 0)),
            pl.BlockSpec((3, cout, 3 * cin), lambda i: (0, 0, 0)),
            pl.BlockSpec((cout, 1), lambda i: (0, 0)),
        ],
        out_specs=pl.BlockSpec((None, nimg, cout, hw), lambda i: (i, 0, 0, 0)),
        scratch_shapes=[pltpu.VMEM((3 * cin, ls), jnp.bfloat16),
                        pltpu.VMEM((3 * cin, ls), jnp.bfloat16),
                        pltpu.VMEM((3 * cin, ls), jnp.bfloat16)],
        compiler_params=pltpu.CompilerParams(
            dimension_semantics=("parallel",),
            vmem_limit_bytes=60 * 1024 * 1024),
    )(x3, wk, b2)
    return out.reshape(n, cout, h, w)


# speedup vs baseline: 1.0073x; 1.0072x over previous
import functools

import jax
import jax.numpy as jnp
from jax import lax
from jax.experimental import pallas as pl
from jax.experimental.pallas import tpu as pltpu


def _conv3x3_kernel(x_ref, w_ref, b_ref, o_ref, s0_ref, s1_ref, *,
                    wdim, hw, base, nimg):
    cin = x_ref.shape[1]
    ls = s0_ref.shape[1]
    scratches = (s0_ref, s1_ref)

    col = lax.broadcasted_iota(jnp.int32, (1, hw), 1) % wdim
    m_r = col == wdim - 1
    m_l = col == 0
    zl0, zl1 = base - 2 * wdim, base + 2 * wdim
    zr0, zr1 = base + hw - 2 * wdim, min(base + hw + 2 * wdim, ls)

    for s in scratches:
        s[:, zl0:zl1] = jnp.zeros((3 * cin, zl1 - zl0), jnp.bfloat16)
        s[:, zr0:zr1] = jnp.zeros((3 * cin, zr1 - zr0), jnp.bfloat16)

    def build(b, s_ref):
        xc = x_ref[b].astype(jnp.bfloat16)
        x_mr = jnp.where(m_r, jnp.bfloat16(0), xc)
        x_ml = jnp.where(m_l, jnp.bfloat16(0), xc)
        srcs = (x_mr, xc, x_ml)
        for kw in range(3):
            p = base - (kw - 1)
            s_ref[kw * cin:(kw + 1) * cin, p:p + hw] = srcs[kw]

    build(0, scratches[0])
    for b in range(nimg):
        s_ref = scratches[b % 2]
        if b + 1 < nimg:
            build(b + 1, scratches[(b + 1) % 2])
        acc = b_ref[...].astype(jnp.float32)
        acc = jnp.broadcast_to(acc, (w_ref.shape[1], hw))
        for kh in range(3):
            t = base + (kh - 1) * wdim
            acc = acc + jnp.dot(w_ref[kh], s_ref[:, t:t + hw],
                                preferred_element_type=jnp.float32)
        o_ref[b] = jnp.maximum(acc, 0.0)


def kernel(x, weight, bias):
    n, cin, h, w = x.shape
    cout = weight.shape[0]
    hw = h * w
    base = 128
    ls = (base + w + 1 + hw + 127) // 128 * 128
    nimg = 8

    x3 = x.reshape(n // nimg, nimg, cin, hw)
    wk = jnp.transpose(weight, (2, 0, 3, 1)).reshape(3, cout, 3 * cin)
    wk = wk.astype(jnp.bfloat16)
    b2 = bias.astype(jnp.float32).reshape(cout, 1)

    body = functools.partial(_conv3x3_kernel, wdim=w, hw=hw, base=base,
                             nimg=nimg)
    out = pl.pallas_call(
        body,
        out_shape=jax.ShapeDtypeStruct((n // nimg, nimg, cout, hw),
                                       jnp.float32),
        grid=(n // nimg,),
        in_specs=[
            pl.BlockSpec((None, nimg, cin, hw), lambda i: (i, 0, 0, 0)),
            pl.BlockSpec((3, cout, 3 * cin), lambda i: (0, 0, 0)),
            pl.BlockSpec((cout, 1), lambda i: (0, 0)),
        ],
        out_specs=pl.BlockSpec((None, nimg, cout, hw), lambda i: (i, 0, 0, 0)),
        scratch_shapes=[pltpu.VMEM((3 * cin, ls), jnp.bfloat16),
                        pltpu.VMEM((3 * cin, ls), jnp.bfloat16)],
        compiler_params=pltpu.CompilerParams(
            dimension_semantics=("parallel",),
            vmem_limit_bytes=60 * 1024 * 1024),
    )(x3, wk, b2)
    return out.reshape(n, cout, h, w)
